# baseline (speedup 1.0000x reference)
import functools

import jax
import jax.numpy as jnp
from jax import lax
from jax.experimental import pallas as pl
from jax.experimental.pallas import tpu as pltpu
from jax.experimental.pallas import tpu_sc as plsc

_N = 100000
_D = 128
_S = 16
_B = 16384
_NC = 2
_NS = 16
_NW = _NC * _NS
_BPW = _B // _NW
_G = _D // 16
_HALF = _BPW // 2
_CH = 128
_NCH = _BPW // _CH
_NT = _N * _S // 128
_TC_ = 64


def _sc_pre(nodes_h, feat_h, self_h, pss_h,
            idx_v, selfbuf, ssbuf, sem_sf):
    wid = lax.axis_index("s") * _NC + lax.axis_index("c")
    base = wid * _BPW
    zeros = jnp.zeros((16,), jnp.float32)

    pltpu.sync_copy(nodes_h.at[pl.ds(base, _BPW)], idx_v)
    pltpu.async_copy(feat_h.at[idx_v.at[pl.ds(0, _HALF)]], selfbuf, sem_sf)

    def _self_ss(ss):
        def body_self(r, ss):
            out = []
            for g in range(_G):
                v = selfbuf[r, pl.ds(g * 16, 16)]
                out.append(ss[g] + v * v)
            return tuple(out)

        return lax.fori_loop(0, _HALF, body_self, ss)

    pltpu.make_async_copy(feat_h.at[pl.ds(0, _HALF)], selfbuf, sem_sf).wait()
    ss_self = _self_ss((zeros,) * _G)
    pltpu.sync_copy(selfbuf, self_h.at[pl.ds(base, _HALF)])
    pltpu.async_copy(feat_h.at[idx_v.at[pl.ds(_HALF, _HALF)]], selfbuf, sem_sf)
    pltpu.make_async_copy(feat_h.at[pl.ds(0, _HALF)], selfbuf, sem_sf).wait()
    ss_self = _self_ss(ss_self)
    pltpu.sync_copy(selfbuf, self_h.at[pl.ds(base + _HALF, _HALF)])

    for g in range(_G):
        ssbuf[pl.ds(g * 16, 16)] = ss_self[g]
    pltpu.sync_copy(ssbuf, pss_h.at[wid])


def _sc_agg(nodes_h, neigh_h, feat_h, mean_h, pss_h,
            idx_v, rowidx_v, samp_rows, sampT_v, accA, accB, ssbuf,
            sem_r0, sem_r1, sem_j0a, sem_j0b, sem_j0c, sem_j0d,
            sem_adda, sem_addb, sem_addc, sem_addd):
    wid = lax.axis_index("s") * _NC + lax.axis_index("c")
    base = wid * _BPW
    lanes = lax.iota(jnp.int32, 16)
    zeros = jnp.zeros((16,), jnp.float32)
    sem_r = (sem_r0, sem_r1)
    sem_j0 = (sem_j0a, sem_j0b)
    sem_j0B = (sem_j0c, sem_j0d)
    sem_add = (sem_adda, sem_addb)
    sem_addB = (sem_addc, sem_addd)

    def _issue_j0(c, p):
        pltpu.async_copy(
            feat_h.at[sampT_v.at[0, pl.ds(c * _CH, _CH)]], accA.at[p], sem_j0[p]
        )
        pltpu.async_copy(
            feat_h.at[sampT_v.at[8, pl.ds(c * _CH, _CH)]], accB.at[p], sem_j0B[p]
        )

    def _issue_adds(c, p):
        for j in range(1, 8):
            pltpu.async_copy(
                feat_h.at[sampT_v.at[j, pl.ds(c * _CH, _CH)]],
                accA.at[p], sem_add[p], add=True,
            )
        for j in range(9, _S):
            pltpu.async_copy(
                feat_h.at[sampT_v.at[j, pl.ds(c * _CH, _CH)]],
                accB.at[p], sem_addB[p], add=True,
            )

    def _wait_j0(p):
        pltpu.make_async_copy(feat_h.at[pl.ds(0, _CH)], accA.at[p], sem_j0[p]).wait()
        pltpu.make_async_copy(feat_h.at[pl.ds(0, _CH)], accB.at[p], sem_j0B[p]).wait()

    def _drain_adds(p):
        for _ in range(1, 8):
            pltpu.make_async_copy(
                feat_h.at[pl.ds(0, _CH)], accA.at[p], sem_add[p]
            ).wait()
        for _ in range(9, _S):
            pltpu.make_async_copy(
                feat_h.at[pl.ds(0, _CH)], accB.at[p], sem_addB[p]
            ).wait()

    pltpu.sync_copy(nodes_h.at[pl.ds(base, _BPW)], idx_v)

    def body_row(c, _):
        v = idx_v[pl.ds(c * 16, 16)]
        rowidx_v[pl.ds(c * 16, 16)] = lax.shift_right_logical(v, 3)
        return 0

    lax.fori_loop(0, _BPW // 16, body_row, 0)

    pltpu.async_copy(
        neigh_h.at[rowidx_v.at[pl.ds(0, _CH)]], samp_rows.at[0], sem_r[0]
    )

    for c in range(_NCH):
        pltpu.make_async_copy(
            neigh_h.at[pl.ds(0, _CH)], samp_rows.at[c & 1], sem_r[c & 1]
        ).wait()
        if c + 1 < _NCH:
            pltpu.async_copy(
                neigh_h.at[rowidx_v.at[pl.ds((c + 1) * _CH, _CH)]],
                samp_rows.at[(c + 1) & 1], sem_r[(c + 1) & 1],
            )

        def body_ext(q, _):
            v = idx_v[pl.ds(c * _CH + q * 16, 16)]
            col0 = (v & 7) * 16
            rows = q * 16 + lanes
            for j in range(_S):
                ids = plsc.load_gather(samp_rows.at[c & 1], [rows, col0 + j])
                sampT_v[j, pl.ds(c * _CH + q * 16, 16)] = ids
            return 0

        lax.fori_loop(0, _CH // 16, body_ext, 0)

        if c == 0:
            _issue_j0(0, 0)
        if c == 1:
            _wait_j0(0)
            _issue_adds(0, 0)
            _issue_j0(1, 1)

    ss_n = (zeros,) * _G
    for c in range(_NCH):
        p = c & 1
        q = 1 - p
        if c + 1 < _NCH:
            _wait_j0(q)
            _issue_adds(c + 1, q)
        _drain_adds(p)

        def body_mean(r, ss):
            out = []
            for g in range(_G):
                sl = pl.ds(g * 16, 16)
                m = (accA[p, r, sl] + accB[p, r, sl]) * (1.0 / _S)
                accA[p, r, sl] = m
                out.append(ss[g] + m * m)
            return tuple(out)

        ss_n = lax.fori_loop(0, _CH, body_mean, ss_n)
        pltpu.sync_copy(accA.at[p], mean_h.at[pl.ds(base + c * _CH, _CH)])
        if c + 2 < _NCH:
            _issue_j0(c + 2, p)

    for g in range(_G):
        ssbuf[pl.ds(g * 16, 16)] = ss_n[g]
    pltpu.sync_copy(ssbuf, pss_h.at[wid])


_pre_call = pl.kernel(
    _sc_pre,
    mesh=plsc.VectorSubcoreMesh(core_axis_name="c", subcore_axis_name="s"),
    compiler_params=pltpu.CompilerParams(needs_layout_passes=False),
    out_type=[
        jax.ShapeDtypeStruct((_B, _D), jnp.float32),
        jax.ShapeDtypeStruct((_NW, _D), jnp.float32),
    ],
    scratch_types=[
        pltpu.VMEM((_BPW,), jnp.int32),
        pltpu.VMEM((_HALF, _D), jnp.float32),
        pltpu.VMEM((_D,), jnp.float32),
        pltpu.SemaphoreType.DMA,
    ],
)

_agg_call = pl.kernel(
    _sc_agg,
    mesh=plsc.VectorSubcoreMesh(core_axis_name="c", subcore_axis_name="s"),
    compiler_params=pltpu.CompilerParams(needs_layout_passes=False),
    out_type=[
        jax.ShapeDtypeStruct((_B, _D), jnp.float32),
        jax.ShapeDtypeStruct((_NW, _D), jnp.float32),
    ],
    scratch_types=[
        pltpu.VMEM((_BPW,), jnp.int32),
        pltpu.VMEM((_BPW,), jnp.int32),
        pltpu.VMEM((2, _CH, 128), jnp.int32),
        pltpu.VMEM((_S, _BPW), jnp.int32),
        pltpu.VMEM((2, _CH, _D), jnp.float32),
        pltpu.VMEM((2, _CH, _D), jnp.float32),
        pltpu.VMEM((_D,), jnp.float32),
        pltpu.SemaphoreType.DMA,
        pltpu.SemaphoreType.DMA,
        pltpu.SemaphoreType.DMA,
        pltpu.SemaphoreType.DMA,
        pltpu.SemaphoreType.DMA,
        pltpu.SemaphoreType.DMA,
        pltpu.SemaphoreType.DMA,
        pltpu.SemaphoreType.DMA,
        pltpu.SemaphoreType.DMA,
        pltpu.SemaphoreType.DMA,
    ],
)

_RB = 4096


def _norm_kernel(pss_s_ref, pss_n_ref, self_ref, mean_ref, out_ref):
    ss_s = jnp.sum(pss_s_ref[...], axis=0)
    ss_n = jnp.sum(pss_n_ref[...], axis=0)
    inv_s = 1.0 / jnp.maximum(jnp.sqrt(ss_s), 1e-12)
    inv_n = 1.0 / jnp.maximum(jnp.sqrt(ss_n), 1e-12)
    out_ref[:, :_D] = self_ref[...] * inv_s[None, :]
    out_ref[:, _D:] = mean_ref[...] * inv_n[None, :]


_norm_call = pl.pallas_call(
    _norm_kernel,
    grid=(_B // _RB,),
    in_specs=[
        pl.BlockSpec((_NW, _D), lambda i: (0, 0)),
        pl.BlockSpec((_NW, _D), lambda i: (0, 0)),
        pl.BlockSpec((_RB, _D), lambda i: (i, 0)),
        pl.BlockSpec((_RB, _D), lambda i: (i, 0)),
    ],
    out_specs=pl.BlockSpec((_RB, 2 * _D), lambda i: (i, 0)),
    out_shape=jax.ShapeDtypeStruct((_B, 2 * _D), jnp.float32),
)


@jax.jit
def kernel(nodes, neigh_idx, features):
    packed = neigh_idx.reshape(_NT, 128)
    self_f, pss_s = _pre_call(nodes, features)
    mean_f, pss_n = _agg_call(nodes, packed, features)
    return _norm_call(pss_s, pss_n, self_f, mean_f)


# speedup vs baseline: 1.0651x; 1.0651x over previous
import functools

import jax
import jax.numpy as jnp
from jax import lax
from jax.experimental import pallas as pl
from jax.experimental.pallas import tpu as pltpu
from jax.experimental.pallas import tpu_sc as plsc

_N = 100000
_D = 128
_S = 16
_B = 16384
_NC = 2
_NS = 16
_NW = _NC * _NS
_BPW = _B // _NW
_G = _D // 16
_HALF = _BPW // 2
_CH = 128
_NCH = _BPW // _CH
_NT = _N * _S // 128
_TC_ = 64


def _sc_pre(nodes_h, feat_h, self_h, pss_h,
            idx_v, selfbuf, ssbuf, sem_sf):
    wid = lax.axis_index("s") * _NC + lax.axis_index("c")
    base = wid * _BPW
    zeros = jnp.zeros((16,), jnp.float32)

    pltpu.sync_copy(nodes_h.at[pl.ds(base, _BPW)], idx_v)
    pltpu.async_copy(feat_h.at[idx_v.at[pl.ds(0, _HALF)]], selfbuf, sem_sf)

    def _self_ss(ss):
        def body_self(r, ss):
            out = []
            for g in range(_G):
                v = selfbuf[r, pl.ds(g * 16, 16)]
                out.append(ss[g] + v * v)
            return tuple(out)

        return lax.fori_loop(0, _HALF, body_self, ss)

    pltpu.make_async_copy(feat_h.at[pl.ds(0, _HALF)], selfbuf, sem_sf).wait()
    ss_self = _self_ss((zeros,) * _G)
    pltpu.sync_copy(selfbuf, self_h.at[pl.ds(base, _HALF)])
    pltpu.async_copy(feat_h.at[idx_v.at[pl.ds(_HALF, _HALF)]], selfbuf, sem_sf)
    pltpu.make_async_copy(feat_h.at[pl.ds(0, _HALF)], selfbuf, sem_sf).wait()
    ss_self = _self_ss(ss_self)
    pltpu.sync_copy(selfbuf, self_h.at[pl.ds(base + _HALF, _HALF)])

    for g in range(_G):
        ssbuf[pl.ds(g * 16, 16)] = ss_self[g]
    pltpu.sync_copy(ssbuf, pss_h.at[wid])


def _sc_agg(nodes_h, neigh_h, feat_h, mean_h, pss_h,
            idx_v, rowidx_v, samp_rows, sampT_v, acc, ssbuf,
            sem_r0, sem_r1, sem_j0a, sem_j0b, sem_j0c,
            sem_adda, sem_addb, sem_addc):
    wid = lax.axis_index("s") * _NC + lax.axis_index("c")
    base = wid * _BPW
    lanes = lax.iota(jnp.int32, 16)
    zeros = jnp.zeros((16,), jnp.float32)
    sem_r = (sem_r0, sem_r1)
    sem_j0 = (sem_j0a, sem_j0b, sem_j0c)
    sem_add = (sem_adda, sem_addb, sem_addc)

    def _issue_j0(c, p):
        pltpu.async_copy(
            feat_h.at[sampT_v.at[0, pl.ds(c * _CH, _CH)]], acc.at[p], sem_j0[p]
        )

    def _issue_adds(c, p):
        for j in range(1, _S):
            pltpu.async_copy(
                feat_h.at[sampT_v.at[j, pl.ds(c * _CH, _CH)]],
                acc.at[p], sem_add[p], add=True,
            )

    def _wait_acc(p, sem):
        pltpu.make_async_copy(feat_h.at[pl.ds(0, _CH)], acc.at[p], sem).wait()

    pltpu.sync_copy(nodes_h.at[pl.ds(base, _BPW)], idx_v)

    def body_row(c, _):
        v = idx_v[pl.ds(c * 16, 16)]
        rowidx_v[pl.ds(c * 16, 16)] = lax.shift_right_logical(v, 3)
        return 0

    lax.fori_loop(0, _BPW // 16, body_row, 0)

    pltpu.async_copy(
        neigh_h.at[rowidx_v.at[pl.ds(0, _CH)]], samp_rows.at[0], sem_r[0]
    )

    for c in range(_NCH):
        pltpu.make_async_copy(
            neigh_h.at[pl.ds(0, _CH)], samp_rows.at[c & 1], sem_r[c & 1]
        ).wait()
        if c + 1 < _NCH:
            pltpu.async_copy(
                neigh_h.at[rowidx_v.at[pl.ds((c + 1) * _CH, _CH)]],
                samp_rows.at[(c + 1) & 1], sem_r[(c + 1) & 1],
            )

        def body_ext(q, _):
            v = idx_v[pl.ds(c * _CH + q * 16, 16)]
            col0 = (v & 7) * 16
            rows = q * 16 + lanes
            for j in range(_S):
                ids = plsc.load_gather(samp_rows.at[c & 1], [rows, col0 + j])
                sampT_v[j, pl.ds(c * _CH + q * 16, 16)] = ids
            return 0

        lax.fori_loop(0, _CH // 16, body_ext, 0)

        if c == 0:
            _issue_j0(0, 0)
        if c == 1:
            _wait_acc(0, sem_j0[0])
            _issue_adds(0, 0)
            _issue_j0(1, 1)

    ss_n = (zeros,) * _G
    for c in range(_NCH):
        p = c % 3
        if c + 2 < _NCH:
            _issue_j0(c + 2, (c + 2) % 3)
        if c + 1 < _NCH:
            q = (c + 1) % 3
            _wait_acc(q, sem_j0[q])
            _issue_adds(c + 1, q)
        for _ in range(1, _S):
            _wait_acc(p, sem_add[p])

        def body_mean(r, ss):
            out = []
            for g in range(_G):
                sl = pl.ds(g * 16, 16)
                m = acc[p, r, sl] * (1.0 / _S)
                acc[p, r, sl] = m
                out.append(ss[g] + m * m)
            return tuple(out)

        ss_n = lax.fori_loop(0, _CH, body_mean, ss_n)
        pltpu.sync_copy(acc.at[p], mean_h.at[pl.ds(base + c * _CH, _CH)])

    for g in range(_G):
        ssbuf[pl.ds(g * 16, 16)] = ss_n[g]
    pltpu.sync_copy(ssbuf, pss_h.at[wid])


_pre_call = pl.kernel(
    _sc_pre,
    mesh=plsc.VectorSubcoreMesh(core_axis_name="c", subcore_axis_name="s"),
    compiler_params=pltpu.CompilerParams(needs_layout_passes=False),
    out_type=[
        jax.ShapeDtypeStruct((_B, _D), jnp.float32),
        jax.ShapeDtypeStruct((_NW, _D), jnp.float32),
    ],
    scratch_types=[
        pltpu.VMEM((_BPW,), jnp.int32),
        pltpu.VMEM((_HALF, _D), jnp.float32),
        pltpu.VMEM((_D,), jnp.float32),
        pltpu.SemaphoreType.DMA,
    ],
)

_agg_call = pl.kernel(
    _sc_agg,
    mesh=plsc.VectorSubcoreMesh(core_axis_name="c", subcore_axis_name="s"),
    compiler_params=pltpu.CompilerParams(needs_layout_passes=False),
    out_type=[
        jax.ShapeDtypeStruct((_B, _D), jnp.float32),
        jax.ShapeDtypeStruct((_NW, _D), jnp.float32),
    ],
    scratch_types=[
        pltpu.VMEM((_BPW,), jnp.int32),
        pltpu.VMEM((_BPW,), jnp.int32),
        pltpu.VMEM((2, _CH, 128), jnp.int32),
        pltpu.VMEM((_S, _BPW), jnp.int32),
        pltpu.VMEM((3, _CH, _D), jnp.float32),
        pltpu.VMEM((_D,), jnp.float32),
        pltpu.SemaphoreType.DMA,
        pltpu.SemaphoreType.DMA,
        pltpu.SemaphoreType.DMA,
        pltpu.SemaphoreType.DMA,
        pltpu.SemaphoreType.DMA,
        pltpu.SemaphoreType.DMA,
        pltpu.SemaphoreType.DMA,
        pltpu.SemaphoreType.DMA,
    ],
)

_RB = 4096


def _norm_half_kernel(pss_ref, half_ref, out_ref):
    ss = jnp.sum(pss_ref[...], axis=0)
    inv = 1.0 / jnp.maximum(jnp.sqrt(ss), 1e-12)
    out_ref[...] = half_ref[...] * inv[None, :]


_norm_self_call = pl.pallas_call(
    _norm_half_kernel,
    grid=(_B // _RB,),
    in_specs=[
        pl.BlockSpec((_NW, _D), lambda i: (0, 0)),
        pl.BlockSpec((_RB, _D), lambda i: (i, 0)),
    ],
    out_specs=pl.BlockSpec((_RB, _D), lambda i: (i, 0)),
    out_shape=jax.ShapeDtypeStruct((_B, 2 * _D), jnp.float32),
)

def _norm_neigh_kernel(pss_ref, half_ref, prev_ref, out_ref):
    ss = jnp.sum(pss_ref[...], axis=0)
    inv = 1.0 / jnp.maximum(jnp.sqrt(ss), 1e-12)
    out_ref[...] = half_ref[...] * inv[None, :]


_norm_neigh_call = pl.pallas_call(
    _norm_neigh_kernel,
    grid=(_B // _RB,),
    in_specs=[
        pl.BlockSpec((_NW, _D), lambda i: (0, 0)),
        pl.BlockSpec((_RB, _D), lambda i: (i, 0)),
        pl.BlockSpec((8, _D), lambda i: (0, 0)),
    ],
    out_specs=pl.BlockSpec((_RB, _D), lambda i: (i, 1)),
    out_shape=jax.ShapeDtypeStruct((_B, 2 * _D), jnp.float32),
    input_output_aliases={2: 0},
)


@jax.jit
def kernel(nodes, neigh_idx, features):
    packed = neigh_idx.reshape(_NT, 128)
    self_f, pss_s = _pre_call(nodes, features)
    mean_f, pss_n = _agg_call(nodes, packed, features)
    out0 = _norm_self_call(pss_s, self_f)
    return _norm_neigh_call(pss_n, mean_f, out0)
